# baseline (speedup 1.0000x reference)
import functools

import jax
import jax.numpy as jnp
from jax import lax
from jax.experimental import pallas as pl
from jax.experimental.pallas import tpu as pltpu
from jax.experimental.pallas import tpu_sc as plsc

_B = 4
_N = 4096
_SCQ = 64
_NT = _N - _SCQ
_BN = _NT // 2

_L = 16
_SB = 2

_NQ0 = _SCQ // 4
_NQ1 = _N // 4


def _cd_body(x1_ref, x2m_ref, d1_ref, d2_ref):
    nb = pl.program_id(1)
    x1 = x1_ref[0]
    x2m = x2m_ref[0]
    x1s = jnp.sum(x1 * x1, axis=1, keepdims=True)
    x2s = 0.25 * jnp.sum(x2m * x2m, axis=0, keepdims=True)
    inner2 = jax.lax.dot_general(
        x1, x2m, dimension_numbers=(((1,), (0,)), ((), ())),
        preferred_element_type=jnp.float32)
    d1_ref[0] = jnp.min(inner2 + x2s, axis=1, keepdims=True) + x1s
    part = jnp.min(inner2 + x1s, axis=0, keepdims=True) + x2s

    @pl.when(nb == 0)
    def _():
        d2_ref[0] = part

    @pl.when(nb > 0)
    def _():
        d2_ref[0] = jnp.minimum(d2_ref[0], part)


def _tc_part(xyz1_head, x2m):
    grid = (_B, _NT // _BN)
    M = _N
    d1, d2 = pl.pallas_call(
        _cd_body,
        grid=grid,
        in_specs=[
            pl.BlockSpec((1, _BN, 3), lambda b, i: (b, i, 0)),
            pl.BlockSpec((1, 3, M), lambda b, i: (b, 0, 0)),
        ],
        out_specs=[
            pl.BlockSpec((1, _BN, 1), lambda b, i: (b, i, 0)),
            pl.BlockSpec((1, 1, M), lambda b, i: (b, 0, 0)),
        ],
        out_shape=[
            jax.ShapeDtypeStruct((_B, _NT, 1), jnp.float32),
            jax.ShapeDtypeStruct((_B, 1, M), jnp.float32),
        ],
        compiler_params=pltpu.CompilerParams(
            dimension_semantics=("parallel", "arbitrary")),
    )(xyz1_head, x2m)
    return d1.reshape(_B, _NT), d2.reshape(_B, M)


def _bf16r(v):
    y = v * jnp.float32(65537.0)
    return y - (y - v)


def _sc_chamfer_body(x1x, x1y, x1z, x2x, x2y, x2z, d1t, d2p,
                     qx, qy, qz, qs, rx, ry, rz, rs, ov):
    nc = 2
    wid = lax.axis_index("s") * nc + lax.axis_index("c")
    b = wid % _B
    dr = (wid // _B) % 2
    c = wid // (_B * 2)

    qo0 = b * _N + _NT + c * _NQ0
    qo1 = b * _N + c * _NQ1
    ro = b * _N

    @pl.when(dr == 0)
    def _():
        pltpu.sync_copy(x1x.at[pl.ds(qo0, _NQ0)], qx.at[pl.ds(0, _NQ0)])
        pltpu.sync_copy(x1y.at[pl.ds(qo0, _NQ0)], qy.at[pl.ds(0, _NQ0)])
        pltpu.sync_copy(x1z.at[pl.ds(qo0, _NQ0)], qz.at[pl.ds(0, _NQ0)])
        pltpu.sync_copy(x2x.at[pl.ds(ro, _N)], rx)
        pltpu.sync_copy(x2y.at[pl.ds(ro, _N)], ry)
        pltpu.sync_copy(x2z.at[pl.ds(ro, _N)], rz)

    @pl.when(dr == 1)
    def _():
        pltpu.sync_copy(x2x.at[pl.ds(qo1, _NQ1)], qx)
        pltpu.sync_copy(x2y.at[pl.ds(qo1, _NQ1)], qy)
        pltpu.sync_copy(x2z.at[pl.ds(qo1, _NQ1)], qz)
        pltpu.sync_copy(x1x.at[pl.ds(ro + _NT, _SCQ)], rx.at[pl.ds(0, _SCQ)])
        pltpu.sync_copy(x1y.at[pl.ds(ro + _NT, _SCQ)], ry.at[pl.ds(0, _SCQ)])
        pltpu.sync_copy(x1z.at[pl.ds(ro + _NT, _SCQ)], rz.at[pl.ds(0, _SCQ)])

    def _rnorm(i, _):
        s = pl.ds(i * _L, _L)
        vx, vy, vz = rx[s], ry[s], rz[s]
        rs[s] = vx * vx + vy * vy + vz * vz
        rx[s] = _bf16r(vx)
        ry[s] = _bf16r(vy)
        rz[s] = _bf16r(vz)
        return 0

    lax.fori_loop(0, _N // _L, _rnorm, 0)

    def _qnorm(i, _):
        s = pl.ds(i * _L, _L)
        vx, vy, vz = qx[s], qy[s], qz[s]
        qs[s] = vx * vx + vy * vy + vz * vz
        qx[s] = -2.0 * _bf16r(vx)
        qy[s] = -2.0 * _bf16r(vy)
        qz[s] = -2.0 * _bf16r(vz)
        return 0

    lax.fori_loop(0, _NQ1 // _L, _qnorm, 0)

    inf16 = jnp.full((_L,), jnp.inf, jnp.float32)

    def _compute(n_sb, n_jc, nsb):
        def _superblock(sb, _):
            base = sb * (nsb * _L)
            mqx = [qx[pl.ds(base + t * _L, _L)] for t in range(nsb)]
            mqy = [qy[pl.ds(base + t * _L, _L)] for t in range(nsb)]
            mqz = [qz[pl.ds(base + t * _L, _L)] for t in range(nsb)]
            nqs = [qs[pl.ds(base + t * _L, _L)] for t in range(nsb)]

            def _jstep(jc, accs):
                s = pl.ds(jc * _L, _L)
                rx16, ry16, rz16, rs16 = rx[s], ry[s], rz[s], rs[s]
                spl = [(rx16[jj], ry16[jj], rz16[jj], rs16[jj])
                       for jj in range(_L)]
                accs = list(accs)
                for t in range(nsb):
                    ds = []
                    for jj in range(_L):
                        rxj, ryj, rzj, rsj = spl[jj]
                        ds.append((nqs[t] + rsj) + mqx[t] * rxj
                                  + mqy[t] * ryj + mqz[t] * rzj)
                    while len(ds) > 1:
                        ds = [jnp.minimum(ds[i], ds[i + 1])
                              for i in range(0, len(ds), 2)]
                    accs[t] = jnp.minimum(accs[t], ds[0])
                return tuple(accs)

            accs = lax.fori_loop(0, n_jc, _jstep, (inf16,) * nsb)
            for t in range(nsb):
                ov[pl.ds(base + t * _L, _L)] = accs[t]
            return 0

        lax.fori_loop(0, n_sb, _superblock, 0)

    @pl.when(dr == 0)
    def _():
        _compute(_NQ0 // _L, _N // _L, 1)
        pltpu.sync_copy(ov.at[pl.ds(0, _NQ0)],
                        d1t.at[pl.ds(b * _SCQ + c * _NQ0, _NQ0)])

    @pl.when(dr == 1)
    def _():
        _compute(_NQ1 // (_SB * _L), _SCQ // _L, _SB)
        pltpu.sync_copy(ov, d2p.at[pl.ds(qo1, _NQ1)])


def _sc_part(x1x, x1y, x1z, x2x, x2y, x2z):
    f32 = jnp.float32
    mesh = plsc.VectorSubcoreMesh(core_axis_name="c", subcore_axis_name="s")
    k = functools.partial(
        pl.kernel,
        mesh=mesh,
        out_type=[
            jax.ShapeDtypeStruct((_B * _SCQ,), f32),
            jax.ShapeDtypeStruct((_B * _N,), f32),
        ],
        scratch_types=[
            pltpu.VMEM((_NQ1,), f32),
            pltpu.VMEM((_NQ1,), f32),
            pltpu.VMEM((_NQ1,), f32),
            pltpu.VMEM((_NQ1,), f32),
            pltpu.VMEM((_N,), f32),
            pltpu.VMEM((_N,), f32),
            pltpu.VMEM((_N,), f32),
            pltpu.VMEM((_N,), f32),
            pltpu.VMEM((_NQ1,), f32),
        ],
    )(_sc_chamfer_body)
    return k(x1x, x1y, x1z, x2x, x2y, x2z)


@jax.jit
def kernel(xyz1, xyz2):
    B, N, _ = xyz1.shape
    M = xyz2.shape[1]
    x1x, x1y, x1z = [xyz1[:, :, k].reshape(-1) for k in range(3)]
    x2x, x2y, x2z = [xyz2[:, :, k].reshape(-1) for k in range(3)]
    d1tail, d2sc = _sc_part(x1x, x1y, x1z, x2x, x2y, x2z)
    x2m = jnp.transpose(-2.0 * xyz2, (0, 2, 1))
    d1head, d2tc = _tc_part(xyz1[:, :_NT, :], x2m)
    d1 = jnp.concatenate([d1head, d1tail.reshape(B, _SCQ)], axis=1)
    d2 = jnp.minimum(d2tc, d2sc.reshape(B, M))
    return d1, d2


# speedup vs baseline: 1.3372x; 1.3372x over previous
import functools

import jax
import jax.numpy as jnp
from jax.experimental import pallas as pl
from jax.experimental.pallas import tpu as pltpu

_BN = 2048


def _cd_body(x1_ref, x2m_ref, d1_ref, d2_ref):
    nb = pl.program_id(1)
    x1 = x1_ref[0]
    x2m = x2m_ref[0]
    x1s = jnp.sum(x1 * x1, axis=1, keepdims=True)
    x2s = 0.25 * jnp.sum(x2m * x2m, axis=0, keepdims=True)
    inner2 = jax.lax.dot_general(
        x1, x2m, dimension_numbers=(((1,), (0,)), ((), ())),
        preferred_element_type=jnp.float32)
    d1_ref[0] = jnp.min(inner2 + x2s, axis=1, keepdims=True) + x1s
    part = jnp.min(inner2 + x1s, axis=0, keepdims=True) + x2s

    @pl.when(nb == 0)
    def _():
        d2_ref[0] = part

    @pl.when(nb > 0)
    def _():
        d2_ref[0] = jnp.minimum(d2_ref[0], part)


@jax.jit
def kernel(xyz1, xyz2):
    B, N, _ = xyz1.shape
    M = xyz2.shape[1]
    x2m = jnp.transpose(-2.0 * xyz2, (0, 2, 1))
    grid = (B, N // _BN)
    d1, d2 = pl.pallas_call(
        _cd_body,
        grid=grid,
        in_specs=[
            pl.BlockSpec((1, _BN, 3), lambda b, i: (b, i, 0)),
            pl.BlockSpec((1, 3, M), lambda b, i: (b, 0, 0)),
        ],
        out_specs=[
            pl.BlockSpec((1, _BN, 1), lambda b, i: (b, i, 0)),
            pl.BlockSpec((1, 1, M), lambda b, i: (b, 0, 0)),
        ],
        out_shape=[
            jax.ShapeDtypeStruct((B, N, 1), jnp.float32),
            jax.ShapeDtypeStruct((B, 1, M), jnp.float32),
        ],
        compiler_params=pltpu.CompilerParams(
            dimension_semantics=("parallel", "arbitrary")),
    )(xyz1, x2m)
    return d1.reshape(B, N), d2.reshape(B, M)
